# Initial kernel scaffold; baseline (speedup 1.0000x reference)
import jax
import jax.numpy as jnp
from jax.experimental import pallas as pl


def kernel(input_idx, type_idx, dpe_idx, E_input, E_type, E_dpe):
    raise NotImplementedError("write your pallas kernel here")


# speedup vs baseline: 8.1626x; 8.1626x over previous
import functools

import jax
import jax.numpy as jnp
from jax import lax
from jax.experimental import pallas as pl
from jax.experimental.pallas import tpu as pltpu
from jax.experimental.pallas import tpu_sc as plsc

D = 128
SEQ = 128
LANES = 16
NUM_WORKERS = 32


def _body(in_idx_hbm, ty_idx_hbm, dp_idx_hbm, ein_hbm, ety_hbm, edp_hbm,
          out_hbm, in_idx_v, ty_idx_v, dp_idx_v, rows_v, out_v, sem,
          *, ev_per_w):
    nc = 2
    wid = lax.axis_index("s") * nc + lax.axis_index("c")

    pltpu.sync_copy(in_idx_hbm.at[wid], in_idx_v)
    pltpu.sync_copy(ty_idx_hbm.at[wid], ty_idx_v)
    pltpu.sync_copy(dp_idx_hbm.at[wid], dp_idx_v)

    nvec = D // LANES
    inv = 1.0 / float(SEQ)

    def per_event(e, carry):
        c0 = pltpu.async_copy(ein_hbm.at[in_idx_v.at[e]],
                              rows_v.at[pl.ds(0, SEQ)], sem)
        c1 = pltpu.async_copy(ety_hbm.at[ty_idx_v.at[e]],
                              rows_v.at[pl.ds(SEQ, SEQ)], sem)
        c2 = pltpu.async_copy(edp_hbm.at[dp_idx_v.at[e]],
                              rows_v.at[pl.ds(2 * SEQ, SEQ)], sem)
        c0.wait()
        c1.wait()
        c2.wait()

        def red(r, acc):
            return tuple(acc[d] + rows_v[r, pl.ds(d * LANES, LANES)]
                         for d in range(nvec))

        zeros = tuple(jnp.zeros((LANES,), jnp.float32) for _ in range(nvec))
        acc = lax.fori_loop(0, 3 * SEQ, red, zeros)
        for d in range(nvec):
            out_v[e, pl.ds(d * LANES, LANES)] = acc[d] * inv
        return carry

    lax.fori_loop(0, ev_per_w, per_event, 0)

    pltpu.sync_copy(out_v, out_hbm.at[wid])


@functools.partial(jax.jit, static_argnames=("num_events",))
def _encode(in_idx, ty_idx, dp_idx, ein, ety, edp, num_events):
    ev_per_w = num_events // NUM_WORKERS
    mesh = plsc.VectorSubcoreMesh(core_axis_name="c", subcore_axis_name="s")
    f = pl.kernel(
        functools.partial(_body, ev_per_w=ev_per_w),
        out_type=jax.ShapeDtypeStruct((NUM_WORKERS, ev_per_w, D), jnp.float32),
        mesh=mesh,
        scratch_types=[
            pltpu.VMEM((ev_per_w, SEQ), jnp.int32),
            pltpu.VMEM((ev_per_w, SEQ), jnp.int32),
            pltpu.VMEM((ev_per_w, SEQ), jnp.int32),
            pltpu.VMEM((3 * SEQ, D), jnp.float32),
            pltpu.VMEM((ev_per_w, D), jnp.float32),
            pltpu.SemaphoreType.DMA,
        ],
    )
    return f(in_idx, ty_idx, dp_idx, ein, ety, edp)


def kernel(input_idx, type_idx, dpe_idx, E_input, E_type, E_dpe):
    b, l, seq = input_idx.shape
    ne = b * l
    ev_per_w = ne // NUM_WORKERS
    shp = (NUM_WORKERS, ev_per_w, seq)
    in_r = input_idx.reshape(shp).astype(jnp.int32)
    ty_r = type_idx.reshape(shp).astype(jnp.int32)
    dp_r = dpe_idx.reshape(shp).astype(jnp.int32)
    out = _encode(in_r, ty_r, dp_r, E_input, E_type, E_dpe, ne)
    return out.reshape(b, l, D)


# speedup vs baseline: 11.9824x; 1.4680x over previous
import functools

import jax
import jax.numpy as jnp
from jax import lax
from jax.experimental import pallas as pl
from jax.experimental.pallas import tpu as pltpu
from jax.experimental.pallas import tpu_sc as plsc

D = 128
SEQ = 128
LANES = 16
NUM_WORKERS = 32
NVEC = D // LANES


def _body(in_idx_hbm, ty_idx_hbm, dp_idx_hbm, ein_hbm, ety_hbm, edp_hbm,
          out_hbm, in_idx_v, ty_idx_v, dp_idx_v, rows_v, out_v, sem0, sem1,
          *, ev_per_w):
    nc = 2
    wid = lax.axis_index("s") * nc + lax.axis_index("c")

    pltpu.sync_copy(in_idx_hbm.at[wid], in_idx_v)
    pltpu.sync_copy(ty_idx_hbm.at[wid], ty_idx_v)
    pltpu.sync_copy(dp_idx_hbm.at[wid], dp_idx_v)

    bufs = (rows_v.at[0], rows_v.at[1])
    sems = (sem0, sem1)
    inv = 1.0 / float(SEQ)

    def issue(e, b):
        pltpu.async_copy(ein_hbm.at[in_idx_v.at[e]],
                         bufs[b].at[pl.ds(0, SEQ)], sems[b])
        pltpu.async_copy(ety_hbm.at[ty_idx_v.at[e]],
                         bufs[b].at[pl.ds(SEQ, SEQ)], sems[b])
        pltpu.async_copy(edp_hbm.at[dp_idx_v.at[e]],
                         bufs[b].at[pl.ds(2 * SEQ, SEQ)], sems[b])

    def drain(b):
        pltpu.make_async_copy(ein_hbm.at[pl.ds(0, 3 * SEQ)], bufs[b],
                              sems[b]).wait()

    issue(0, 0)

    @pl.loop(0, ev_per_w, step=2)
    def _events(e0):
        for b in range(2):
            e = e0 + b
            drain(b)

            @pl.when(e + 1 < ev_per_w)
            def _():
                issue(e + 1, 1 - b)

            buf = bufs[b]

            def red(r, acc):
                return tuple(acc[d] + buf[r, pl.ds(d * LANES, LANES)]
                             for d in range(NVEC))

            zeros = tuple(jnp.zeros((LANES,), jnp.float32)
                          for _ in range(NVEC))
            acc = plsc.parallel_loop(0, 3 * SEQ, unroll=4, carry=zeros)(red)
            for d in range(NVEC):
                out_v[e, pl.ds(d * LANES, LANES)] = acc[d] * inv

    pltpu.sync_copy(out_v, out_hbm.at[wid])


@functools.partial(jax.jit, static_argnames=("num_events",))
def _encode(in_idx, ty_idx, dp_idx, ein, ety, edp, num_events):
    ev_per_w = num_events // NUM_WORKERS
    mesh = plsc.VectorSubcoreMesh(core_axis_name="c", subcore_axis_name="s")
    f = pl.kernel(
        functools.partial(_body, ev_per_w=ev_per_w),
        out_type=jax.ShapeDtypeStruct((NUM_WORKERS, ev_per_w, D), jnp.float32),
        mesh=mesh,
        scratch_types=[
            pltpu.VMEM((ev_per_w, SEQ), jnp.int32),
            pltpu.VMEM((ev_per_w, SEQ), jnp.int32),
            pltpu.VMEM((ev_per_w, SEQ), jnp.int32),
            pltpu.VMEM((2, 3 * SEQ, D), jnp.float32),
            pltpu.VMEM((ev_per_w, D), jnp.float32),
            pltpu.SemaphoreType.DMA,
            pltpu.SemaphoreType.DMA,
        ],
    )
    return f(in_idx, ty_idx, dp_idx, ein, ety, edp)


def kernel(input_idx, type_idx, dpe_idx, E_input, E_type, E_dpe):
    b, l, seq = input_idx.shape
    ne = b * l
    ev_per_w = ne // NUM_WORKERS
    shp = (NUM_WORKERS, ev_per_w, seq)
    in_r = input_idx.reshape(shp).astype(jnp.int32)
    ty_r = type_idx.reshape(shp).astype(jnp.int32)
    dp_r = dpe_idx.reshape(shp).astype(jnp.int32)
    out = _encode(in_r, ty_r, dp_r, E_input, E_type, E_dpe, ne)
    return out.reshape(b, l, D)
